# baseline (speedup 1.0000x reference)
import functools

import jax
import jax.numpy as jnp
from jax import lax
from jax.experimental import pallas as pl
from jax.experimental.pallas import tpu as pltpu
from jax.experimental.pallas import tpu_sc as plsc

_N = 10000
_D = 128
_NC = 2
_NS = 16
_CHUNK = 128
_NPAD = 10240
_ROWS = _NPAD // _NS


def _sc_mesh():
    return plsc.VectorSubcoreMesh(core_axis_name="c", subcore_axis_name="s")


def _make_hist(C):
    @functools.partial(
        pl.kernel,
        out_type=jax.ShapeDtypeStruct((_NC, _NPAD, _D), jnp.float32),
        mesh=_sc_mesh(),
        scratch_types=[
            pltpu.VMEM((C, _CHUNK), jnp.int32),
            pltpu.VMEM((_CHUNK, _D), jnp.float32),
            pltpu.VMEM_SHARED((_NPAD, _D), jnp.float32),
        ],
    )
    def hist(dst_hbm, ones_hbm, zeros_hbm, out_hbm, dstv, onesv, acc):
        c = lax.axis_index("c")
        s = lax.axis_index("s")
        pltpu.sync_copy(dst_hbm.at[c, s], dstv)
        pltpu.sync_copy(ones_hbm, onesv)
        pltpu.sync_copy(zeros_hbm.at[pl.ds(s * _ROWS, _ROWS)],
                        acc.at[pl.ds(s * _ROWS, _ROWS)])
        plsc.subcore_barrier()

        @pl.loop(0, C)
        def _(j):
            pltpu.sync_copy(onesv, acc.at[dstv.at[j]], add=True)

        plsc.subcore_barrier()
        pltpu.sync_copy(acc.at[pl.ds(s * _ROWS, _ROWS)],
                        out_hbm.at[c, pl.ds(s * _ROWS, _ROWS)])

    return hist


_IB = 8


def _make_agg(C):
    assert C % _IB == 0

    @functools.partial(
        pl.kernel,
        out_type=jax.ShapeDtypeStruct((_NC, _NPAD, _D), jnp.float32),
        mesh=_sc_mesh(),
        scratch_types=[
            pltpu.VMEM((C, _CHUNK), jnp.int32),
            pltpu.VMEM((C, _CHUNK), jnp.int32),
            pltpu.VMEM((_CHUNK, _D), jnp.float32),
            pltpu.VMEM_SHARED((_NPAD, _D), jnp.float32),
        ],
    )
    def agg(z_hbm, src_hbm, dst_hbm, zeros_hbm, out_hbm, srcv, dstv, rowsv,
            acc):
        c = lax.axis_index("c")
        s = lax.axis_index("s")
        pltpu.sync_copy(src_hbm.at[c, s], srcv)
        pltpu.sync_copy(dst_hbm.at[c, s], dstv)
        pltpu.sync_copy(zeros_hbm.at[pl.ds(s * _ROWS, _ROWS)],
                        acc.at[pl.ds(s * _ROWS, _ROWS)])
        plsc.subcore_barrier()

        @pl.loop(0, C)
        def _(j):
            pltpu.sync_copy(z_hbm.at[srcv.at[j]], rowsv)
            pltpu.sync_copy(rowsv, acc.at[dstv.at[j]], add=True)

        plsc.subcore_barrier()
        pltpu.sync_copy(acc.at[pl.ds(s * _ROWS, _ROWS)],
                        out_hbm.at[c, pl.ds(s * _ROWS, _ROWS)])

    return agg


def _tc_matmul(x, W1):
    def body(x_ref, w_ref, out_ref):
        out_ref[...] = jnp.dot(
            x_ref[...], w_ref[...], preferred_element_type=jnp.float32)

    return pl.pallas_call(
        body, out_shape=jax.ShapeDtypeStruct((_N, _D), jnp.float32)
    )(x, W1)


def _tc_scale(h, cntp):
    def body(h_ref, cnt_ref, out_ref):
        cnt = cnt_ref[0] + cnt_ref[1]
        dinv = lax.rsqrt(cnt + 1.0)[:_N, 0:1]
        out_ref[...] = h_ref[...] * dinv

    return pl.pallas_call(
        body, out_shape=jax.ShapeDtypeStruct((_N, _D), jnp.float32)
    )(h, cntp)


def _tc_mid(Sp, zt, cntp, b, W2):
    def body(sp_ref, zt_ref, cnt_ref, b_ref, w_ref, out_ref):
        cnt = cnt_ref[0] + cnt_ref[1]
        dinv = lax.rsqrt(cnt + 1.0)[:_N, 0:1]
        S = sp_ref[0, :_N, :] + sp_ref[1, :_N, :] + zt_ref[...]
        h = jnp.maximum(S * dinv + b_ref[...], 0.0)
        out_ref[...] = jnp.dot(
            h, w_ref[...], preferred_element_type=jnp.float32) * dinv

    return pl.pallas_call(
        body, out_shape=jax.ShapeDtypeStruct((_N, _D), jnp.float32)
    )(Sp, zt, cntp, b, W2)


def _tc_last(Sp, zt, cntp, b, Wl, bl):
    def body(sp_ref, zt_ref, cnt_ref, b_ref, wl_ref, bl_ref, out_ref):
        cnt = cnt_ref[0] + cnt_ref[1]
        dinv = lax.rsqrt(cnt + 1.0)[:_N, 0:1]
        S = sp_ref[0, :_N, :] + sp_ref[1, :_N, :] + zt_ref[...]
        h = jnp.maximum(S * dinv + b_ref[...], 0.0)
        out_ref[...] = jnp.dot(
            h, wl_ref[...], preferred_element_type=jnp.float32) + bl_ref[...]

    return pl.pallas_call(
        body, out_shape=jax.ShapeDtypeStruct((_N, Wl.shape[1]), jnp.float32)
    )(Sp, zt, cntp, b, Wl, bl)


def kernel(x, adjacency, W1, b1, W2, b2, Wl, bl):
    E = adjacency.shape[1]
    per = _NC * _NS * _CHUNK
    C = -(-E // per)
    C = -(-C // _IB) * _IB
    pad = C * per - E

    src = adjacency[0].astype(jnp.int32)
    dst = adjacency[1].astype(jnp.int32)
    fill = jnp.arange(pad, dtype=jnp.int32)
    src = jnp.concatenate([src, (fill * 37) % _N])
    dst = jnp.concatenate([dst, _N + (fill % (_NPAD - _N))])
    src4 = src.reshape(_NC, _NS, C, _CHUNK)
    dst4 = dst.reshape(_NC, _NS, C, _CHUNK)

    zerosD = jnp.zeros((_NPAD, _D), jnp.float32)
    onesD = jnp.ones((_CHUNK, _D), jnp.float32)

    hist = _make_hist(C)
    agg = _make_agg(C)

    cntp = hist(dst4, onesD, zerosD)
    h1 = _tc_matmul(x, W1)
    z1t = _tc_scale(h1, cntp)
    S1p = agg(z1t, src4, dst4, zerosD)
    z2t = _tc_mid(S1p, z1t, cntp, b1.reshape(1, _D), W2)
    S2p = agg(z2t, src4, dst4, zerosD)
    out = _tc_last(S2p, z2t, cntp, b2.reshape(1, _D), Wl, bl.reshape(1, 2))
    return out


# speedup vs baseline: 1.0052x; 1.0052x over previous
import functools

import jax
import jax.numpy as jnp
from jax import lax
from jax.experimental import pallas as pl
from jax.experimental.pallas import tpu as pltpu
from jax.experimental.pallas import tpu_sc as plsc

_N = 10000
_D = 128
_NC = 2
_NS = 16
_CHUNK = 128
_NPAD = 10240
_ROWS = _NPAD // _NS


def _sc_mesh():
    return plsc.VectorSubcoreMesh(core_axis_name="c", subcore_axis_name="s")


def _make_hist(C):
    @functools.partial(
        pl.kernel,
        out_type=jax.ShapeDtypeStruct((_NC, _NPAD, _D), jnp.float32),
        mesh=_sc_mesh(),
        scratch_types=[
            pltpu.VMEM((C, _CHUNK), jnp.int32),
            pltpu.VMEM((_CHUNK, _D), jnp.float32),
            pltpu.VMEM_SHARED((_NPAD, _D), jnp.float32),
        ],
    )
    def hist(dst_hbm, ones_hbm, zeros_hbm, out_hbm, dstv, onesv, acc):
        c = lax.axis_index("c")
        s = lax.axis_index("s")
        pltpu.sync_copy(dst_hbm.at[c, s], dstv)
        pltpu.sync_copy(ones_hbm, onesv)
        pltpu.sync_copy(zeros_hbm.at[pl.ds(s * _ROWS, _ROWS)],
                        acc.at[pl.ds(s * _ROWS, _ROWS)])
        plsc.subcore_barrier()

        @pl.loop(0, C)
        def _(j):
            pltpu.sync_copy(onesv, acc.at[dstv.at[j]], add=True)

        plsc.subcore_barrier()
        pltpu.sync_copy(acc.at[pl.ds(s * _ROWS, _ROWS)],
                        out_hbm.at[c, pl.ds(s * _ROWS, _ROWS)])

    return hist


_IB = 8


def _make_agg(C):
    assert C % _IB == 0

    @functools.partial(
        pl.kernel,
        out_type=jax.ShapeDtypeStruct((_NC, _NPAD, _D), jnp.float32),
        mesh=_sc_mesh(),
        scratch_types=[
            pltpu.VMEM((C, _CHUNK), jnp.int32),
            pltpu.VMEM((C, _CHUNK), jnp.int32),
            pltpu.VMEM((_CHUNK, _D), jnp.float32),
            pltpu.VMEM_SHARED((_NPAD, _D), jnp.float32),
        ],
    )
    def agg(z_hbm, src_hbm, dst_hbm, zeros_hbm, out_hbm, srcv, dstv, rowsv,
            acc):
        c = lax.axis_index("c")
        s = lax.axis_index("s")
        pltpu.sync_copy(src_hbm.at[c, s], srcv)
        pltpu.sync_copy(dst_hbm.at[c, s], dstv)
        pltpu.sync_copy(zeros_hbm.at[pl.ds(s * _ROWS, _ROWS)],
                        acc.at[pl.ds(s * _ROWS, _ROWS)])
        plsc.subcore_barrier()

        @pl.loop(0, C)
        def _(j):
            pltpu.sync_copy(z_hbm.at[srcv.at[j]], rowsv)
            pltpu.sync_copy(rowsv, acc.at[dstv.at[j]], add=True)

        plsc.subcore_barrier()
        pltpu.sync_copy(acc.at[pl.ds(s * _ROWS, _ROWS)],
                        out_hbm.at[c, pl.ds(s * _ROWS, _ROWS)])

    return agg


def _tc_first(x, W1, cntp):
    def body(x_ref, w_ref, cnt_ref, out_ref):
        cnt = cnt_ref[0] + cnt_ref[1]
        dinv = lax.rsqrt(cnt + 1.0)[:_N, 0:1]
        h = jnp.dot(x_ref[...], w_ref[...], preferred_element_type=jnp.float32)
        out_ref[...] = h * dinv

    return pl.pallas_call(
        body, out_shape=jax.ShapeDtypeStruct((_N, _D), jnp.float32)
    )(x, W1, cntp)


def _tc_mid(Sp, zt, cntp, b, W2):
    def body(sp_ref, zt_ref, cnt_ref, b_ref, w_ref, out_ref):
        cnt = cnt_ref[0] + cnt_ref[1]
        dinv = lax.rsqrt(cnt + 1.0)[:_N, 0:1]
        S = sp_ref[0, :_N, :] + sp_ref[1, :_N, :] + zt_ref[...]
        h = jnp.maximum(S * dinv + b_ref[...], 0.0)
        out_ref[...] = jnp.dot(
            h, w_ref[...], preferred_element_type=jnp.float32) * dinv

    return pl.pallas_call(
        body, out_shape=jax.ShapeDtypeStruct((_N, _D), jnp.float32)
    )(Sp, zt, cntp, b, W2)


def _tc_last(Sp, zt, cntp, b, Wl, bl):
    def body(sp_ref, zt_ref, cnt_ref, b_ref, wl_ref, bl_ref, out_ref):
        cnt = cnt_ref[0] + cnt_ref[1]
        dinv = lax.rsqrt(cnt + 1.0)[:_N, 0:1]
        S = sp_ref[0, :_N, :] + sp_ref[1, :_N, :] + zt_ref[...]
        h = jnp.maximum(S * dinv + b_ref[...], 0.0)
        out_ref[...] = jnp.dot(
            h, wl_ref[...], preferred_element_type=jnp.float32) + bl_ref[...]

    return pl.pallas_call(
        body, out_shape=jax.ShapeDtypeStruct((_N, Wl.shape[1]), jnp.float32)
    )(Sp, zt, cntp, b, Wl, bl)


def kernel(x, adjacency, W1, b1, W2, b2, Wl, bl):
    E = adjacency.shape[1]
    per = _NC * _NS * _CHUNK
    C = -(-E // per)
    C = -(-C // _IB) * _IB
    pad = C * per - E

    src = adjacency[0].astype(jnp.int32)
    dst = adjacency[1].astype(jnp.int32)
    fill = jnp.arange(pad, dtype=jnp.int32)
    src = jnp.concatenate([src, (fill * 37) % _N])
    dst = jnp.concatenate([dst, _N + (fill % (_NPAD - _N))])
    src4 = src.reshape(_NC, _NS, C, _CHUNK)
    dst4 = dst.reshape(_NC, _NS, C, _CHUNK)

    zerosD = jnp.zeros((_NPAD, _D), jnp.float32)
    onesD = jnp.ones((_CHUNK, _D), jnp.float32)

    hist = _make_hist(C)
    agg = _make_agg(C)

    cntp = hist(dst4, onesD, zerosD)[:, :, 0:8]
    z1t = _tc_first(x, W1, cntp)
    S1p = agg(z1t, src4, dst4, zerosD)
    z2t = _tc_mid(S1p, z1t, cntp, b1.reshape(1, _D), W2)
    S2p = agg(z2t, src4, dst4, zerosD)
    out = _tc_last(S2p, z2t, cntp, b2.reshape(1, _D), Wl, bl.reshape(1, 2))
    return out


# speedup vs baseline: 1.2211x; 1.2148x over previous
import functools

import jax
import jax.numpy as jnp
from jax import lax
from jax.experimental import pallas as pl
from jax.experimental.pallas import tpu as pltpu
from jax.experimental.pallas import tpu_sc as plsc

_N = 10000
_D = 128
_NC = 2
_NS = 16
_CHUNK = 128
_NPAD = 10240
_ROWS = _NPAD // _NS


def _sc_mesh():
    return plsc.VectorSubcoreMesh(core_axis_name="c", subcore_axis_name="s")


def _make_hist(C):
    @functools.partial(
        pl.kernel,
        out_type=jax.ShapeDtypeStruct((_NC, _NPAD, _D), jnp.float32),
        mesh=_sc_mesh(),
        scratch_types=[
            pltpu.VMEM((C, _CHUNK), jnp.int32),
            pltpu.VMEM((_CHUNK, _D), jnp.float32),
            pltpu.VMEM_SHARED((_NPAD, _D), jnp.float32),
        ],
    )
    def hist(dst_hbm, ones_hbm, zeros_hbm, out_hbm, dstv, onesv, acc):
        c = lax.axis_index("c")
        s = lax.axis_index("s")
        pltpu.sync_copy(dst_hbm.at[c, s], dstv)
        pltpu.sync_copy(ones_hbm, onesv)
        pltpu.sync_copy(zeros_hbm.at[pl.ds(s * _ROWS, _ROWS)],
                        acc.at[pl.ds(s * _ROWS, _ROWS)])
        plsc.subcore_barrier()

        @pl.loop(0, C)
        def _(j):
            pltpu.sync_copy(onesv, acc.at[dstv.at[j]], add=True)

        plsc.subcore_barrier()
        pltpu.sync_copy(acc.at[pl.ds(s * _ROWS, _ROWS)],
                        out_hbm.at[c, pl.ds(s * _ROWS, _ROWS)])

    return hist


_IB = 8


def _make_agg(C):
    assert C % _IB == 0

    @functools.partial(
        pl.kernel,
        out_type=jax.ShapeDtypeStruct((_NC, _NPAD, _D), jnp.float32),
        mesh=_sc_mesh(),
        scratch_types=[
            pltpu.VMEM((C // 2, _CHUNK), jnp.int32),
            pltpu.VMEM((C // 2, _CHUNK), jnp.int32),
            pltpu.VMEM_SHARED((_NPAD, _D), jnp.float32),
        ],
    )
    def agg(z_hbm, src_hbm, dst_hbm, zeros_hbm, out_hbm, srcv, dstv, acc):
        HC = C // 2
        c = lax.axis_index("c")
        s = lax.axis_index("s")
        pltpu.sync_copy(zeros_hbm.at[pl.ds(s * _ROWS, _ROWS)],
                        acc.at[pl.ds(s * _ROWS, _ROWS)])
        plsc.subcore_barrier()

        def pipelined(buf0, buf1, sem0):
            @pl.loop(0, 2)
            def _(h):
                pltpu.sync_copy(src_hbm.at[c, s, pl.ds(h * HC, HC)], srcv)
                pltpu.sync_copy(dst_hbm.at[c, s, pl.ds(h * HC, HC)], dstv)

                @pl.loop(0, HC, step=2)
                def _(j):
                    g0 = pltpu.async_copy(z_hbm.at[srcv.at[j]], buf0, sem0)
                    g1 = pltpu.async_copy(z_hbm.at[srcv.at[j + 1]], buf1,
                                          sem0)
                    g0.wait()
                    pltpu.sync_copy(buf0, acc.at[dstv.at[j]], add=True)
                    g1.wait()
                    pltpu.sync_copy(buf1, acc.at[dstv.at[j + 1]], add=True)

        pl.run_scoped(pipelined,
                      pltpu.VMEM((_CHUNK, _D), jnp.float32),
                      pltpu.VMEM((_CHUNK, _D), jnp.float32),
                      pltpu.SemaphoreType.DMA)

        plsc.subcore_barrier()
        pltpu.sync_copy(acc.at[pl.ds(s * _ROWS, _ROWS)],
                        out_hbm.at[c, pl.ds(s * _ROWS, _ROWS)])

    return agg


def _tc_first(x, W1, cntp):
    def body(x_ref, w_ref, cnt_ref, out_ref):
        cnt = cnt_ref[0] + cnt_ref[1]
        dinv = lax.rsqrt(cnt + 1.0)[:_N, 0:1]
        h = jnp.dot(x_ref[...], w_ref[...], preferred_element_type=jnp.float32)
        out_ref[...] = h * dinv

    return pl.pallas_call(
        body, out_shape=jax.ShapeDtypeStruct((_N, _D), jnp.float32)
    )(x, W1, cntp)


def _tc_mid(Sp, zt, cntp, b, W2):
    def body(sp_ref, zt_ref, cnt_ref, b_ref, w_ref, out_ref):
        cnt = cnt_ref[0] + cnt_ref[1]
        dinv = lax.rsqrt(cnt + 1.0)[:_N, 0:1]
        S = sp_ref[0, :_N, :] + sp_ref[1, :_N, :] + zt_ref[...]
        h = jnp.maximum(S * dinv + b_ref[...], 0.0)
        out_ref[...] = jnp.dot(
            h, w_ref[...], preferred_element_type=jnp.float32) * dinv

    return pl.pallas_call(
        body, out_shape=jax.ShapeDtypeStruct((_N, _D), jnp.float32)
    )(Sp, zt, cntp, b, W2)


def _tc_last(Sp, zt, cntp, b, Wl, bl):
    def body(sp_ref, zt_ref, cnt_ref, b_ref, wl_ref, bl_ref, out_ref):
        cnt = cnt_ref[0] + cnt_ref[1]
        dinv = lax.rsqrt(cnt + 1.0)[:_N, 0:1]
        S = sp_ref[0, :_N, :] + sp_ref[1, :_N, :] + zt_ref[...]
        h = jnp.maximum(S * dinv + b_ref[...], 0.0)
        out_ref[...] = jnp.dot(
            h, wl_ref[...], preferred_element_type=jnp.float32) + bl_ref[...]

    return pl.pallas_call(
        body, out_shape=jax.ShapeDtypeStruct((_N, Wl.shape[1]), jnp.float32)
    )(Sp, zt, cntp, b, Wl, bl)


def kernel(x, adjacency, W1, b1, W2, b2, Wl, bl):
    E = adjacency.shape[1]
    per = _NC * _NS * _CHUNK
    C = -(-E // per)
    C = -(-C // _IB) * _IB
    pad = C * per - E

    src = adjacency[0].astype(jnp.int32)
    dst = adjacency[1].astype(jnp.int32)
    fill = jnp.arange(pad, dtype=jnp.int32)
    src = jnp.concatenate([src, (fill * 37) % _N])
    dst = jnp.concatenate([dst, _N + (fill % (_NPAD - _N))])
    src4 = src.reshape(_NC, _NS, C, _CHUNK)
    dst4 = dst.reshape(_NC, _NS, C, _CHUNK)

    zerosD = jnp.zeros((_NPAD, _D), jnp.float32)
    onesD = jnp.ones((_CHUNK, _D), jnp.float32)

    hist = _make_hist(C)
    agg = _make_agg(C)

    cntp = hist(dst4, onesD, zerosD)[:, :, 0:8]
    z1t = _tc_first(x, W1, cntp)
    S1p = agg(z1t, src4, dst4, zerosD)
    z2t = _tc_mid(S1p, z1t, cntp, b1.reshape(1, _D), W2)
    S2p = agg(z2t, src4, dst4, zerosD)
    out = _tc_last(S2p, z2t, cntp, b2.reshape(1, _D), Wl, bl.reshape(1, 2))
    return out
